# baseline (speedup 1.0000x reference)
import functools

import jax
import jax.numpy as jnp
from jax import lax
from jax.experimental import pallas as pl
from jax.experimental.pallas import tpu as pltpu
from jax.experimental.pallas import tpu_sc as plsc

B, L, N, H = 32, 8192, 64, 128
NUM_REL = 8
NC, NS = 2, 16
TB = 4
M = TB * N


def _sc_gather_body(emb_hbm, ids_hbm, out_hbm, idx_v, rows_v, sem):
    wid = lax.axis_index("s") * NC + lax.axis_index("c")
    pltpu.sync_copy(ids_hbm.at[wid], idx_v)
    for i in range(N // 16):
        sl = pl.ds(i * 16, 16)
        idx_v[sl] = idx_v[sl] + wid * L
    pltpu.async_copy(emb_hbm.at[idx_v], rows_v, sem).wait()
    pltpu.sync_copy(rows_v, out_hbm.at[wid])


def _sc_gather(emb_flat, num_ids):
    mesh = plsc.VectorSubcoreMesh(core_axis_name="c", subcore_axis_name="s")
    k = functools.partial(
        pl.kernel,
        mesh=mesh,
        out_type=jax.ShapeDtypeStruct((B, N, H), jnp.float32),
        scratch_types=[
            pltpu.VMEM((N,), jnp.int32),
            pltpu.VMEM((N, H), jnp.float32),
            pltpu.SemaphoreType.DMA,
        ],
    )(_sc_gather_body)
    return k(emb_flat, num_ids)


def _dense_body(x_ref, ncol_ref, nrow_ref, icol_ref, irow_ref,
                wa_ref, ba_ref, wft_ref, bf_ref, wrt_ref, out_ref):
    x = x_ref[0]
    ncol = ncol_ref[0]
    nrow = nrow_ref[0]
    icol = icol_ref[0]
    irow = irow_ref[0]

    lin = jnp.sum(x * wa_ref[...], axis=1, keepdims=True) + ba_ref[...]
    a = 1.0 / (1.0 + jnp.exp(-lin))

    gt = jnp.where(ncol > nrow, 1.0, 0.0)
    rel = 4.0 * gt + 2.0 * icol + irow
    ii = lax.broadcasted_iota(jnp.int32, (M, M), 0)
    jj = lax.broadcasted_iota(jnp.int32, (M, M), 1)
    valid = (ii != jj) & ((ii >> 6) == (jj >> 6))

    msg = jnp.zeros((M, H), jnp.float32)
    for r in range(NUM_REL):
        sel = jnp.where((rel == float(r)) & valid, 1.0, 0.0)
        u = jnp.dot(x, wrt_ref[r], preferred_element_type=jnp.float32) * a
        msg = msg + jnp.dot(sel, u, preferred_element_type=jnp.float32)

    y = (jnp.dot(x, wft_ref[...], preferred_element_type=jnp.float32)
         + bf_ref[...] + msg * (1.0 / (N - 1)))
    out_ref[0] = jnp.maximum(y, 0.0)


def _dense(x, numbers, isr, wa_row, ba, wft, bf, wrt, interpret=False):
    g = B // TB
    ncol = numbers.reshape(g, M, 1)
    nrow = numbers.reshape(g, 1, M)
    icol = isr.reshape(g, M, 1)
    irow = isr.reshape(g, 1, M)
    out = pl.pallas_call(
        _dense_body,
        grid=(g,),
        in_specs=[
            pl.BlockSpec((1, M, H), lambda i: (i, 0, 0)),
            pl.BlockSpec((1, M, 1), lambda i: (i, 0, 0)),
            pl.BlockSpec((1, 1, M), lambda i: (i, 0, 0)),
            pl.BlockSpec((1, M, 1), lambda i: (i, 0, 0)),
            pl.BlockSpec((1, 1, M), lambda i: (i, 0, 0)),
            pl.BlockSpec((1, H), lambda i: (0, 0)),
            pl.BlockSpec((1, 1), lambda i: (0, 0)),
            pl.BlockSpec((H, H), lambda i: (0, 0)),
            pl.BlockSpec((1, H), lambda i: (0, 0)),
            pl.BlockSpec((NUM_REL, H, H), lambda i: (0, 0, 0)),
        ],
        out_specs=pl.BlockSpec((1, M, H), lambda i: (i, 0, 0)),
        out_shape=jax.ShapeDtypeStruct((g, M, H), jnp.float32),
        interpret=interpret,
    )(x.reshape(g, M, H), ncol, nrow, icol, irow, wa_row, ba, wft, bf, wrt)
    return out.reshape(B, N, H)


def kernel(word_emb, num_ids, is_response, numbers, W_alpha, b_alpha, W_f, b_f, W_r):
    init_emb = _sc_gather(word_emb.reshape(B * L, H), num_ids)
    return _dense(
        init_emb,
        numbers,
        is_response.astype(jnp.float32),
        W_alpha.reshape(1, H),
        b_alpha.reshape(1, 1),
        W_f.T,
        b_f.reshape(1, H),
        jnp.swapaxes(W_r, 1, 2),
    )


# speedup vs baseline: 1.2209x; 1.2209x over previous
import functools

import jax
import jax.numpy as jnp
from jax import lax
from jax.experimental import pallas as pl
from jax.experimental.pallas import tpu as pltpu
from jax.experimental.pallas import tpu_sc as plsc

B, L, N, H = 32, 8192, 64, 128
NUM_REL = 8
NC, NS = 2, 16
TB = 4
M = TB * N


def _sc_gather_body(emb_hbm, ids_hbm, out_hbm, idx_v, rows_v, sem):
    wid = lax.axis_index("s") * NC + lax.axis_index("c")
    pltpu.sync_copy(ids_hbm.at[wid], idx_v)
    for i in range(N // 16):
        sl = pl.ds(i * 16, 16)
        idx_v[sl] = idx_v[sl] + wid * L
    pltpu.async_copy(emb_hbm.at[idx_v], rows_v, sem).wait()
    pltpu.sync_copy(rows_v, out_hbm.at[wid])


def _sc_gather(emb_flat, num_ids):
    mesh = plsc.VectorSubcoreMesh(core_axis_name="c", subcore_axis_name="s")
    k = functools.partial(
        pl.kernel,
        mesh=mesh,
        out_type=jax.ShapeDtypeStruct((B, N, H), jnp.float32),
        scratch_types=[
            pltpu.VMEM((N,), jnp.int32),
            pltpu.VMEM((N, H), jnp.float32),
            pltpu.SemaphoreType.DMA,
        ],
    )(_sc_gather_body)
    return k(emb_flat, num_ids)


def _nt(m, w):
    return lax.dot_general(m, w, (((1,), (1,)), ((), ())),
                           preferred_element_type=jnp.float32)


def _dense_body(x_ref, ncol_ref, nrow_ref, rcol_ref,
                wa_ref, ba_ref, wf_ref, bf_ref, wr_ref, out_ref):
    x = x_ref[0]
    ncol = ncol_ref[0]
    nrow = nrow_ref[0]
    rcol = rcol_ref[0]

    lin = jnp.sum(x * wa_ref[...], axis=1, keepdims=True) + ba_ref[...]
    a = (1.0 / (N - 1)) / (1.0 + jnp.exp(-lin))

    ax = x * a
    x1 = jnp.where(rcol > 0.5, ax, 0.0)
    x0 = ax - x1

    ii = lax.broadcasted_iota(jnp.int32, (M, M), 0)
    jj = lax.broadcasted_iota(jnp.int32, (M, M), 1)
    valid = jnp.where((ii != jj) & ((ii >> 6) == (jj >> 6)), 1.0, 0.0)
    G = jnp.where(ncol > nrow, valid, 0.0)
    Gc = valid - G

    A0 = jnp.dot(G, x0, preferred_element_type=jnp.float32)
    A1 = jnp.dot(G, x1, preferred_element_type=jnp.float32)
    B0 = jnp.dot(Gc, x0, preferred_element_type=jnp.float32)
    B1 = jnp.dot(Gc, x1, preferred_element_type=jnp.float32)
    msg0 = _nt(A0, wr_ref[4]) + _nt(A1, wr_ref[5]) + _nt(B0, wr_ref[0]) + _nt(B1, wr_ref[1])
    msg1 = _nt(A0, wr_ref[6]) + _nt(A1, wr_ref[7]) + _nt(B0, wr_ref[2]) + _nt(B1, wr_ref[3])
    msg = jnp.where(rcol > 0.5, msg1, msg0)

    y = _nt(x, wf_ref[...]) + bf_ref[...] + msg
    out_ref[0] = jnp.maximum(y, 0.0)


def _dense(x, numbers, isr, wa_row, ba, wf, bf, wr, interpret=False):
    g = B // TB
    out = pl.pallas_call(
        _dense_body,
        grid=(g,),
        in_specs=[
            pl.BlockSpec((1, M, H), lambda i: (i, 0, 0)),
            pl.BlockSpec((1, M, 1), lambda i: (i, 0, 0)),
            pl.BlockSpec((1, 1, M), lambda i: (i, 0, 0)),
            pl.BlockSpec((1, M, 1), lambda i: (i, 0, 0)),
            pl.BlockSpec((1, H), lambda i: (0, 0)),
            pl.BlockSpec((1, 1), lambda i: (0, 0)),
            pl.BlockSpec((H, H), lambda i: (0, 0)),
            pl.BlockSpec((1, H), lambda i: (0, 0)),
            pl.BlockSpec((NUM_REL, H, H), lambda i: (0, 0, 0)),
        ],
        out_specs=pl.BlockSpec((1, M, H), lambda i: (i, 0, 0)),
        out_shape=jax.ShapeDtypeStruct((g, M, H), jnp.float32),
        interpret=interpret,
    )(x.reshape(g, M, H), numbers.reshape(g, M, 1), numbers.reshape(g, 1, M),
      isr.reshape(g, M, 1), wa_row, ba, wf, bf, wr)
    return out.reshape(B, N, H)


def kernel(word_emb, num_ids, is_response, numbers, W_alpha, b_alpha, W_f, b_f, W_r):
    init_emb = _sc_gather(word_emb.reshape(B * L, H), num_ids)
    return _dense(
        init_emb,
        numbers,
        is_response.astype(jnp.float32),
        W_alpha.reshape(1, H),
        b_alpha.reshape(1, 1),
        W_f,
        b_f.reshape(1, H),
        W_r,
    )


# speedup vs baseline: 1.3391x; 1.0968x over previous
import functools

import jax
import jax.numpy as jnp
from jax import lax
from jax.experimental import pallas as pl
from jax.experimental.pallas import tpu as pltpu
from jax.experimental.pallas import tpu_sc as plsc

B, L, N, H = 32, 8192, 64, 128
NUM_REL = 8
NC, NS = 2, 16
TB = 4
M = TB * N


def _sc_gather_body(emb_hbm, ids_hbm, out_hbm, idx_v, rows_v, sem):
    wid = lax.axis_index("s") * NC + lax.axis_index("c")
    pltpu.sync_copy(ids_hbm.at[wid], idx_v)
    for i in range(N // 16):
        sl = pl.ds(i * 16, 16)
        idx_v[sl] = idx_v[sl] + wid * L
    pltpu.async_copy(emb_hbm.at[idx_v], rows_v, sem).wait()
    pltpu.sync_copy(rows_v, out_hbm.at[wid])


def _sc_gather(emb_flat, num_ids):
    mesh = plsc.VectorSubcoreMesh(core_axis_name="c", subcore_axis_name="s")
    k = functools.partial(
        pl.kernel,
        mesh=mesh,
        out_type=jax.ShapeDtypeStruct((B, N, H), jnp.float32),
        scratch_types=[
            pltpu.VMEM((N,), jnp.int32),
            pltpu.VMEM((N, H), jnp.float32),
            pltpu.SemaphoreType.DMA,
        ],
    )(_sc_gather_body)
    return k(emb_flat, num_ids)


def _nt(m, w):
    return lax.dot_general(m, w, (((1,), (1,)), ((), ())),
                           preferred_element_type=jnp.float32)


GS = 2
CH = B // (TB * GS)


def _dense_body(x_ref, ncol_ref, nrow_ref, rcol_ref,
                wa_ref, ba_ref, wf_ref, bf_ref, wr_ref, out_ref):
    ii = lax.broadcasted_iota(jnp.int32, (M, M), 0)
    jj = lax.broadcasted_iota(jnp.int32, (M, M), 1)
    valid = jnp.where((ii != jj) & ((ii >> 6) == (jj >> 6)), 1.0, 0.0)

    for c in range(CH):
        x = x_ref[0, c]
        ncol = ncol_ref[0, c]
        nrow = nrow_ref[0, c]
        rcol = rcol_ref[0, c]

        lin = jnp.sum(x * wa_ref[...], axis=1, keepdims=True) + ba_ref[...]
        a = (1.0 / (N - 1)) / (1.0 + jnp.exp(-lin))

        ax = x * a
        x1 = jnp.where(rcol > 0.5, ax, 0.0)
        x0 = ax - x1

        G = jnp.where(ncol > nrow, valid, 0.0)

        A0 = jnp.dot(G, x0, preferred_element_type=jnp.float32)
        A1 = jnp.dot(G, x1, preferred_element_type=jnp.float32)
        cs0 = jnp.sum(x0.reshape(TB, N, H), axis=1, keepdims=True)
        cs1 = jnp.sum(x1.reshape(TB, N, H), axis=1, keepdims=True)
        S0 = jnp.broadcast_to(cs0, (TB, N, H)).reshape(M, H)
        S1 = jnp.broadcast_to(cs1, (TB, N, H)).reshape(M, H)
        B0 = S0 - x0 - A0
        B1 = S1 - x1 - A1
        msg0 = _nt(A0, wr_ref[4]) + _nt(A1, wr_ref[5]) + _nt(B0, wr_ref[0]) + _nt(B1, wr_ref[1])
        msg1 = _nt(A0, wr_ref[6]) + _nt(A1, wr_ref[7]) + _nt(B0, wr_ref[2]) + _nt(B1, wr_ref[3])
        msg = jnp.where(rcol > 0.5, msg1, msg0)

        y = _nt(x, wf_ref[...]) + bf_ref[...] + msg
        out_ref[0, c] = jnp.maximum(y, 0.0)


def _dense(x, numbers, isr, wa_row, ba, wf, bf, wr, interpret=False):
    out = pl.pallas_call(
        _dense_body,
        grid=(GS,),
        in_specs=[
            pl.BlockSpec((1, CH, M, H), lambda i: (i, 0, 0, 0)),
            pl.BlockSpec((1, CH, M, 1), lambda i: (i, 0, 0, 0)),
            pl.BlockSpec((1, CH, 1, M), lambda i: (i, 0, 0, 0)),
            pl.BlockSpec((1, CH, M, 1), lambda i: (i, 0, 0, 0)),
            pl.BlockSpec((1, H), lambda i: (0, 0)),
            pl.BlockSpec((1, 1), lambda i: (0, 0)),
            pl.BlockSpec((H, H), lambda i: (0, 0)),
            pl.BlockSpec((1, H), lambda i: (0, 0)),
            pl.BlockSpec((NUM_REL, H, H), lambda i: (0, 0, 0)),
        ],
        out_specs=pl.BlockSpec((1, CH, M, H), lambda i: (i, 0, 0, 0)),
        out_shape=jax.ShapeDtypeStruct((GS, CH, M, H), jnp.float32),
        interpret=interpret,
    )(x.reshape(GS, CH, M, H), numbers.reshape(GS, CH, M, 1),
      numbers.reshape(GS, CH, 1, M), isr.reshape(GS, CH, M, 1),
      wa_row, ba, wf, bf, wr)
    return out.reshape(B, N, H)


def kernel(word_emb, num_ids, is_response, numbers, W_alpha, b_alpha, W_f, b_f, W_r):
    init_emb = _sc_gather(word_emb.reshape(B * L, H), num_ids)
    return _dense(
        init_emb,
        numbers,
        is_response.astype(jnp.float32),
        W_alpha.reshape(1, H),
        b_alpha.reshape(1, 1),
        W_f,
        b_f.reshape(1, H),
        W_r,
    )
